# baseline (speedup 1.0000x reference)
import jax
import jax.numpy as jnp
import numpy as np
from jax.experimental import pallas as pl
from jax.experimental.pallas import tpu as pltpu

_ALPHA = 3.0
_MIN_DIFF = 0.1
_N = 4096

_B = 1024
_G = _N // _B
_NT = _G * (_G + 1) // 2

_RBS = np.array([r for r in range(_G) for c in range(r, _G)], dtype=np.int32)
_CBS = np.array([c for r in range(_G) for c in range(r, _G)], dtype=np.int32)

_LN2 = float(np.log(2.0))
_LOG2E = float(np.log2(np.e))


def _vreg_sum(x):
    return jnp.sum(x, axis=0, keepdims=True)


def _tri_tile(rbs_ref, cbs_ref, p_col_ref, t_col_ref, p_row_ref, t_row_ref,
              out_ref, ld_ref, lo_ref, cd_ref, co_ref, reg_ref):
    k = pl.program_id(0)
    rb = rbs_ref[k]
    cb = cbs_ref[k]

    @pl.when(k == 0)
    def _init():
        z = jnp.zeros((1, _B), jnp.float32)
        ld_ref[...] = z
        lo_ref[...] = z
        cd_ref[...] = z
        co_ref[...] = z
        reg_ref[...] = jnp.zeros((1, 1), jnp.float32)

    p_i = p_col_ref[...]
    t_i = t_col_ref[...]
    p_j = p_row_ref[...]
    t_j = t_row_ref[...]

    d = t_i - t_j
    dps = p_i - p_j
    c = jnp.abs(d)
    mask = c > _MIN_DIFF
    md = jnp.where(mask, d, 0.0)
    e = jnp.exp2(-jnp.abs(dps))
    term = jnp.maximum(-md * dps, 0.0) + jnp.abs(md) * jnp.log2(1.0 + e)
    maskf = jnp.where(mask, 1.0, 0.0)
    partial = _vreg_sum(term)
    pcnt = _vreg_sum(maskf)

    @pl.when(cb == rb)
    def _acc_diag():
        ld_ref[...] += partial
        cd_ref[...] += pcnt
        err = p_i * _LN2 - t_i
        reg_ref[...] += jnp.sum(err * err, keepdims=True)

    @pl.when(cb != rb)
    def _acc_off():
        lo_ref[...] += partial
        co_ref[...] += pcnt

    @pl.when(k == _NT - 1)
    def _combine():
        loss_sum = (jnp.sum(ld_ref[...]) + 2.0 * jnp.sum(lo_ref[...])) * _LN2
        cnt = jnp.sum(cd_ref[...]) + 2.0 * jnp.sum(co_ref[...])
        reg = reg_ref[0, 0] * (1.0 / _N)
        pair_mean = loss_sum / jnp.maximum(cnt, 1.0)
        total = jnp.where(cnt > 0, reg + _ALPHA * pair_mean, reg)
        out_ref[...] = total.reshape(1, 1)


@jax.jit
def kernel(pred, target):
    ps = pred.reshape(_N) * jnp.float32(_LOG2E)
    p = ps.reshape(_N, 1)
    t = target.reshape(_N, 1)
    p_row = ps.reshape(1, _N)
    t_row = target.reshape(1, _N)

    grid_spec = pltpu.PrefetchScalarGridSpec(
        num_scalar_prefetch=2,
        grid=(_NT,),
        in_specs=[
            pl.BlockSpec((_B, 1), lambda k, rbs, cbs: (rbs[k], 0)),
            pl.BlockSpec((_B, 1), lambda k, rbs, cbs: (rbs[k], 0)),
            pl.BlockSpec((1, _B), lambda k, rbs, cbs: (0, cbs[k])),
            pl.BlockSpec((1, _B), lambda k, rbs, cbs: (0, cbs[k])),
        ],
        out_specs=pl.BlockSpec((1, 1), lambda k, rbs, cbs: (0, 0)),
        scratch_shapes=[
            pltpu.VMEM((1, _B), jnp.float32),
            pltpu.VMEM((1, _B), jnp.float32),
            pltpu.VMEM((1, _B), jnp.float32),
            pltpu.VMEM((1, _B), jnp.float32),
            pltpu.VMEM((1, 1), jnp.float32),
        ],
    )
    out = pl.pallas_call(
        _tri_tile,
        grid_spec=grid_spec,
        out_shape=jax.ShapeDtypeStruct((1, 1), jnp.float32),
    )(jnp.asarray(_RBS), jnp.asarray(_CBS), p, t, p_row, t_row)

    return out[0, 0]


# speedup vs baseline: 1.0052x; 1.0052x over previous
import jax
import jax.numpy as jnp
import numpy as np
from jax.experimental import pallas as pl
from jax.experimental.pallas import tpu as pltpu

_ALPHA = 3.0
_MIN_DIFF = 0.1
_N = 4096

_B = 1024
_G = _N // _B
_NT = _G * (_G + 1) // 2
_RC = 256

_RBS = np.array([r for r in range(_G) for c in range(r, _G)], dtype=np.int32)
_CBS = np.array([c for r in range(_G) for c in range(r, _G)], dtype=np.int32)

_LN2 = float(np.log(2.0))
_LOG2E = float(np.log2(np.e))


def _vreg_sum(x):
    return jnp.sum(x, axis=0, keepdims=True)


def _tri_tile(rbs_ref, cbs_ref, p_col_ref, t_col_ref, p_row_ref, t_row_ref,
              out_ref, ld_ref, lo_ref, cd_ref, co_ref, reg_ref):
    k = pl.program_id(0)
    rb = rbs_ref[k]
    cb = cbs_ref[k]

    @pl.when(k == 0)
    def _init():
        z = jnp.zeros((1, _B), jnp.float32)
        ld_ref[...] = z
        lo_ref[...] = z
        cd_ref[...] = z
        co_ref[...] = z
        reg_ref[...] = jnp.zeros((1, 1), jnp.float32)

    p_j = p_row_ref[...]
    t_j = t_row_ref[...]

    partial = jnp.zeros((1, _B), jnp.float32)
    pcnt = jnp.zeros((1, _B), jnp.float32)
    for i0 in range(0, _B, _RC):
        p_i = p_col_ref[i0:i0 + _RC, :]
        t_i = t_col_ref[i0:i0 + _RC, :]
        d = t_i - t_j
        dps = p_i - p_j
        c = jnp.abs(d)
        mask = c > _MIN_DIFF
        md = jnp.where(mask, d, 0.0)
        e = jnp.exp2(-jnp.abs(dps))
        term = jnp.maximum(-md * dps, 0.0) + jnp.abs(md) * jnp.log2(1.0 + e)
        maskf = jnp.where(mask, 1.0, 0.0)
        partial = partial + _vreg_sum(term)
        pcnt = pcnt + _vreg_sum(maskf)

    @pl.when(cb == rb)
    def _acc_diag():
        ld_ref[...] += partial
        cd_ref[...] += pcnt
        err = p_col_ref[...] * _LN2 - t_col_ref[...]
        reg_ref[...] += jnp.sum(err * err, keepdims=True)

    @pl.when(cb != rb)
    def _acc_off():
        lo_ref[...] += partial
        co_ref[...] += pcnt

    @pl.when(k == _NT - 1)
    def _combine():
        loss_sum = (jnp.sum(ld_ref[...]) + 2.0 * jnp.sum(lo_ref[...])) * _LN2
        cnt = jnp.sum(cd_ref[...]) + 2.0 * jnp.sum(co_ref[...])
        reg = reg_ref[0, 0] * (1.0 / _N)
        pair_mean = loss_sum / jnp.maximum(cnt, 1.0)
        total = jnp.where(cnt > 0, reg + _ALPHA * pair_mean, reg)
        out_ref[...] = total.reshape(1, 1)


@jax.jit
def kernel(pred, target):
    ps = pred.reshape(_N) * jnp.float32(_LOG2E)
    p = ps.reshape(_N, 1)
    t = target.reshape(_N, 1)
    p_row = ps.reshape(1, _N)
    t_row = target.reshape(1, _N)

    grid_spec = pltpu.PrefetchScalarGridSpec(
        num_scalar_prefetch=2,
        grid=(_NT,),
        in_specs=[
            pl.BlockSpec((_B, 1), lambda k, rbs, cbs: (rbs[k], 0)),
            pl.BlockSpec((_B, 1), lambda k, rbs, cbs: (rbs[k], 0)),
            pl.BlockSpec((1, _B), lambda k, rbs, cbs: (0, cbs[k])),
            pl.BlockSpec((1, _B), lambda k, rbs, cbs: (0, cbs[k])),
        ],
        out_specs=pl.BlockSpec((1, 1), lambda k, rbs, cbs: (0, 0)),
        scratch_shapes=[
            pltpu.VMEM((1, _B), jnp.float32),
            pltpu.VMEM((1, _B), jnp.float32),
            pltpu.VMEM((1, _B), jnp.float32),
            pltpu.VMEM((1, _B), jnp.float32),
            pltpu.VMEM((1, 1), jnp.float32),
        ],
    )
    out = pl.pallas_call(
        _tri_tile,
        grid_spec=grid_spec,
        out_shape=jax.ShapeDtypeStruct((1, 1), jnp.float32),
    )(jnp.asarray(_RBS), jnp.asarray(_CBS), p, t, p_row, t_row)

    return out[0, 0]
